# baseline (speedup 1.0000x reference)
import functools

import jax
import jax.numpy as jnp
from jax import lax
from jax.experimental import pallas as pl
from jax.experimental.pallas import tpu as pltpu
from jax.experimental.pallas import tpu_sc as plsc

B, T = 4096, 200
D = 64
CHUNK = 128
NBUF = 2

_cache = {}


def _build():
    if "k" in _cache:
        return _cache["k"]
    info = plsc.get_sparse_core_info()
    NC, NS = info.num_cores, info.num_subcores
    NW = NC * NS
    TT, BT = T // 8, B // CHUNK
    blocks_per_w = (TT * BT) // NW
    chunks_per_w = blocks_per_w * 8
    n_groups = chunks_per_w // NBUF
    assert n_groups % 2 == 0 and n_groups >= 4
    mesh = plsc.VectorSubcoreMesh(core_axis_name="c", subcore_axis_name="s")

    @functools.partial(
        pl.kernel,
        mesh=mesh,
        compiler_params=pltpu.CompilerParams(
            use_tc_tiling_on_sc=False, needs_layout_passes=False
        ),
        out_type=jax.ShapeDtypeStruct((T, D // 8, B // CHUNK, 8, CHUNK), jnp.float32),
        scratch_types=[
            pltpu.VMEM((blocks_per_w, 8, CHUNK), jnp.int32),
            pltpu.VMEM((2, NBUF, CHUNK, D), jnp.float32),
            pltpu.VMEM((2, NBUF, D // 8, 8, CHUNK), jnp.float32),
            pltpu.SemaphoreType.DMA,
            pltpu.SemaphoreType.DMA,
            pltpu.SemaphoreType.DMA,
            pltpu.SemaphoreType.DMA,
        ],
    )
    def emb(idx_hbm, table_hbm, out_hbm, idx_v, rows_v, tr_v,
            g0sem, g1sem, s0sem, s1sem):
        wid = lax.axis_index("s") * NC + lax.axis_index("c")
        fb0 = wid * blocks_per_w
        gsems = [g0sem, g1sem]
        ssems = [s0sem, s1sem]
        iota16 = lax.iota(jnp.int32, 16)
        pltpu.sync_copy(idx_hbm.at[pl.ds(fb0, blocks_per_w)], idx_v)

        def chunk_ids(gi, j):
            c = gi * NBUF + j
            return c // 8, c % 8

        def start_gathers(gi, s):
            for j in range(NBUF):
                fbl, ti = chunk_ids(gi, j)
                pltpu.async_copy(
                    table_hbm.at[idx_v.at[fbl, ti]],
                    rows_v.at[s, j],
                    gsems[s],
                )

        def wait_gathers(s):
            for j in range(NBUF):
                pltpu.make_async_copy(
                    table_hbm.at[idx_v.at[0, j]],
                    rows_v.at[s, j],
                    gsems[s],
                ).wait()

        def transpose(s):
            for j in range(NBUF):
                src = rows_v.at[s, j]
                dst = tr_v.at[s, j]

                def dbody(d0, carry, src=src, dst=dst):
                    for du in range(4):
                        d = d0 * 4 + du
                        for bg in range(8):
                            vals = plsc.load_gather(
                                src,
                                [iota16 + bg * 16, jnp.full((16,), d, jnp.int32)],
                            )
                            dst[d // 8, d % 8, pl.ds(bg * 16, 16)] = vals
                    return carry

                lax.fori_loop(0, D // 4, dbody, 0)

        def start_stores(gi, s):
            for j in range(NBUF):
                fbl, ti = chunk_ids(gi, j)
                fb = fb0 + fbl
                tt = fb // BT
                bt = fb - tt * BT
                t = tt * 8 + ti
                pltpu.async_copy(
                    tr_v.at[s, j],
                    out_hbm.at[t, :, bt],
                    ssems[s],
                )

        def wait_stores(s):
            for j in range(NBUF):
                pltpu.make_async_copy(
                    tr_v.at[s, j],
                    out_hbm.at[0, :, 0],
                    ssems[s],
                ).wait()

        start_gathers(0, 0)
        start_gathers(1, 1)
        wait_gathers(0)
        transpose(0)
        start_stores(0, 0)
        start_gathers(2, 0)
        wait_gathers(1)
        transpose(1)
        start_stores(1, 1)

        def pair(p, carry):
            ge = 2 * p + 2
            start_gathers(ge + 1, 1)
            wait_gathers(0)
            wait_stores(0)
            transpose(0)
            start_stores(ge, 0)
            go = ge + 1
            start_gathers(go + 1, 0)
            wait_gathers(1)
            wait_stores(1)
            transpose(1)
            start_stores(go, 1)
            return carry

        lax.fori_loop(0, (n_groups - 4) // 2, pair, 0)

        start_gathers(n_groups - 1, 1)
        wait_gathers(0)
        wait_stores(0)
        transpose(0)
        start_stores(n_groups - 2, 0)
        wait_gathers(1)
        wait_stores(1)
        transpose(1)
        start_stores(n_groups - 1, 1)
        wait_stores(0)
        wait_stores(1)

    _cache["k"] = emb
    return emb


def kernel(idx_texts, table):
    idx_k = (
        idx_texts.astype(jnp.int32)
        .reshape(B // CHUNK, CHUNK, T // 8, 8)
        .transpose(2, 0, 3, 1)
        .reshape((T // 8) * (B // CHUNK), 8, CHUNK)
    )
    out5 = _build()(idx_k, table)
    return out5.transpose(2, 4, 0, 1, 3).reshape(B, T, D)


# speedup vs baseline: 2.7237x; 2.7237x over previous
import functools

import jax
import jax.numpy as jnp
from jax import lax
from jax.experimental import pallas as pl
from jax.experimental.pallas import tpu as pltpu
from jax.experimental.pallas import tpu_sc as plsc

B, T = 4096, 200
D = 64
CHUNK = 128
NBUF = 4

_cache = {}


def _build():
    if "k" in _cache:
        return _cache["k"]
    info = plsc.get_sparse_core_info()
    NC, NS = info.num_cores, info.num_subcores
    NW = NC * NS
    TT, BT = T // 8, B // CHUNK
    blocks_per_w = (TT * BT) // NW
    chunks_per_w = blocks_per_w * 8
    n_groups = chunks_per_w // NBUF
    assert n_groups % 2 == 0 and n_groups >= 4
    mesh = plsc.VectorSubcoreMesh(core_axis_name="c", subcore_axis_name="s")

    @functools.partial(
        pl.kernel,
        mesh=mesh,
        compiler_params=pltpu.CompilerParams(use_tc_tiling_on_sc=False),
        out_type=jax.ShapeDtypeStruct((B, T, 2 * D), jnp.float32),
        scratch_types=[
            pltpu.VMEM((blocks_per_w, 8, CHUNK), jnp.int32),
            pltpu.VMEM((2, NBUF, CHUNK, D), jnp.float32),
            pltpu.SemaphoreType.DMA,
            pltpu.SemaphoreType.DMA,
            pltpu.SemaphoreType.DMA,
            pltpu.SemaphoreType.DMA,
        ],
    )
    def emb(idx_hbm, table_hbm, out_hbm, idx_v, rows_v, g0sem, g1sem, s0sem, s1sem):
        wid = lax.axis_index("s") * NC + lax.axis_index("c")
        fb0 = wid * blocks_per_w
        gsems = [g0sem, g1sem]
        ssems = [s0sem, s1sem]
        pltpu.sync_copy(idx_hbm.at[pl.ds(fb0, blocks_per_w)], idx_v)

        def start_gathers(gi, s):
            fbl = gi // 2
            ti0 = (gi % 2) * NBUF
            for j in range(NBUF):
                pltpu.async_copy(
                    table_hbm.at[idx_v.at[fbl, ti0 + j]],
                    rows_v.at[s, j],
                    gsems[s],
                )

        def wait_gathers(s):
            for j in range(NBUF):
                pltpu.make_async_copy(
                    table_hbm.at[idx_v.at[0, j]],
                    rows_v.at[s, j],
                    gsems[s],
                ).wait()

        def start_stores(gi, s):
            fb = fb0 + gi // 2
            tt = fb // BT
            bt = fb - tt * BT
            b0 = bt * CHUNK
            t0 = tt * 8 + (gi % 2) * NBUF
            for j in range(NBUF):
                pltpu.async_copy(
                    rows_v.at[s, j],
                    out_hbm.at[pl.ds(b0, CHUNK), t0 + j, pl.ds(0, D)],
                    ssems[s],
                )

        def wait_stores(s):
            for j in range(NBUF):
                pltpu.make_async_copy(
                    rows_v.at[s, j],
                    out_hbm.at[pl.ds(0, CHUNK), 0, pl.ds(0, D)],
                    ssems[s],
                ).wait()

        start_gathers(0, 0)
        start_gathers(1, 1)
        wait_gathers(0)
        start_stores(0, 0)

        def pair(p, carry):
            go = 2 * p + 1
            wait_stores(0)
            start_gathers(go + 1, 0)
            wait_gathers(1)
            start_stores(go, 1)
            ge = go + 1
            wait_stores(1)
            start_gathers(ge + 1, 1)
            wait_gathers(0)
            start_stores(ge, 0)
            return carry

        lax.fori_loop(0, (n_groups - 2) // 2, pair, 0)

        wait_gathers(1)
        start_stores(n_groups - 1, 1)
        wait_stores(0)
        wait_stores(1)

    _cache["k"] = emb
    return emb


def kernel(idx_texts, table):
    idx_k = (
        idx_texts.astype(jnp.int32)
        .reshape(B // CHUNK, CHUNK, T // 8, 8)
        .transpose(2, 0, 3, 1)
        .reshape((T // 8) * (B // CHUNK), 8, CHUNK)
    )
    from jax.experimental import layout as jl
    table_l = jl.with_layout_constraint(
        table, jl.Layout((0, 1), tiling=((8, 64),))
    )
    out2 = _build()(idx_k, table_l)
    return out2.reshape(B * T, 2 * D)[:, :D].reshape(B, T, D)
